# Initial kernel scaffold; baseline (speedup 1.0000x reference)
import jax
import jax.numpy as jnp
from jax.experimental import pallas as pl


def kernel(input, adj, h0, weight, Wg, bg, We, be, lamda, alpha, l):
    raise NotImplementedError("write your pallas kernel here")


# speedup vs baseline: 1.4675x; 1.4675x over previous
import jax
import jax.numpy as jnp
from jax.experimental import pallas as pl
from jax.experimental.pallas import tpu as pltpu


def _fused_kernel(scal_ref, x_ref, adj_ref, h0_ref, w_ref, wg_ref, bg_ref,
                  we_ref, be_ref, out_ref):
    theta = scal_ref[0, 0]
    alpha = scal_ref[0, 1]
    e_num = we_ref.shape[0]

    hi = jnp.dot(adj_ref[...], x_ref[...], preferred_element_type=jnp.float32)
    support = (1.0 - alpha) * hi + alpha * h0_ref[...]
    sw = jnp.dot(support, w_ref[...], preferred_element_type=jnp.float32)
    out_lin = theta * sw + (1.0 - theta) * support

    logits = jnp.dot(out_lin, wg_ref[...],
                     preferred_element_type=jnp.float32) + bg_ref[...]
    idx = jax.lax.broadcasted_iota(jnp.int32, logits.shape, 1)
    v1 = jnp.max(logits, axis=-1, keepdims=True)
    a1 = jnp.min(jnp.where(logits == v1, idx, e_num), axis=-1, keepdims=True)
    masked = jnp.where(idx == a1, -jnp.inf, logits)
    v2 = jnp.max(masked, axis=-1, keepdims=True)
    a2 = jnp.min(jnp.where(masked == v2, idx, e_num), axis=-1, keepdims=True)
    t = jnp.exp(v2 - v1)
    denom = 1.0 + t
    wts = ((idx == a1).astype(jnp.float32)
           + t * (idx == a2).astype(jnp.float32)) / denom

    acc = jnp.zeros_like(out_lin)
    for e in range(e_num):
        h_e = jnp.dot(out_lin, we_ref[e],
                      preferred_element_type=jnp.float32) + be_ref[e:e + 1, :]
        acc = acc + wts[:, e:e + 1] * h_e
    out_ref[...] = acc


def kernel(input, adj, h0, weight, Wg, bg, We, be, lamda, alpha, l):
    n, d = input.shape
    e_num = We.shape[0]
    bm = next((b for b in (400, 200, 100, 50, 25, 10, 8) if n % b == 0), n)

    theta = jnp.log(lamda / l + 1.0)
    scal = jnp.stack([jnp.asarray(theta, jnp.float32),
                      jnp.asarray(alpha, jnp.float32)]).reshape(1, 2)
    bg2 = bg.reshape(1, e_num).astype(jnp.float32)

    return pl.pallas_call(
        _fused_kernel,
        grid=(n // bm,),
        in_specs=[
            pl.BlockSpec((1, 2), lambda i: (0, 0)),
            pl.BlockSpec((n, d), lambda i: (0, 0)),
            pl.BlockSpec((bm, n), lambda i: (i, 0)),
            pl.BlockSpec((bm, d), lambda i: (i, 0)),
            pl.BlockSpec((d, d), lambda i: (0, 0)),
            pl.BlockSpec((d, e_num), lambda i: (0, 0)),
            pl.BlockSpec((1, e_num), lambda i: (0, 0)),
            pl.BlockSpec((e_num, d, d), lambda i: (0, 0, 0)),
            pl.BlockSpec((e_num, d), lambda i: (0, 0)),
        ],
        out_specs=pl.BlockSpec((bm, d), lambda i: (i, 0)),
        out_shape=jax.ShapeDtypeStruct((n, d), jnp.float32),
        compiler_params=pltpu.CompilerParams(
            dimension_semantics=("parallel",)),
    )(scal, input, adj, h0, weight, Wg, bg2, We, be)
